# baseline (speedup 1.0000x reference)
import jax
import jax.numpy as jnp
from jax.experimental import pallas as pl
from jax.experimental.pallas import tpu as pltpu

BATCH = 4096
DIM = 128
BLK = 512
NBLK = BATCH // BLK


def _loss_body(a_ref, b_ref, lab_i_ref, lab_j_ref, out_ref, acc_ref):
    i = pl.program_id(0)
    j = pl.program_id(1)

    @pl.when((i == 0) & (j == 0))
    def _init():
        acc_ref[0] = 0.0
        acc_ref[1] = 0.0
        acc_ref[2] = 0.0
        acc_ref[3] = 0.0

    @pl.when(j >= i)
    def _compute():
        a_bf = a_ref[...].astype(jnp.bfloat16)
        b_bf = b_ref[...].astype(jnp.bfloat16)
        g = jax.lax.dot_general(
            a_bf, b_bf, dimension_numbers=(((1,), (1,)), ((), ())),
            preferred_element_type=jnp.float32)
        a = a_bf.astype(jnp.float32)
        b = b_bf.astype(jnp.float32)
        an = jnp.sum(a * a, axis=1, keepdims=True)
        bn = jnp.sum(b * b, axis=1)[None, :]
        sq = an - 2.0 * g + bn
        dist = jnp.sqrt(jnp.maximum(sq, 1e-16))

        pos_mask = lab_i_ref[...] == lab_j_ref[...]
        zero = jnp.zeros_like(dist)
        pos_sum = jnp.sum(jnp.where(pos_mask, dist, zero))
        pos_cnt = jnp.sum(pos_mask.astype(jnp.float32))
        neg_l = jnp.maximum(1.0 - dist, 0.0)
        neg_sum = jnp.sum(jnp.where(pos_mask, zero, neg_l))
        neg_cnt = jnp.sum(jnp.where(pos_mask | (dist >= 1.0), zero,
                                    jnp.ones_like(dist)))

        w = jnp.where(i == j, 1.0, 2.0)
        acc_ref[0] = acc_ref[0] + w * pos_sum
        acc_ref[1] = acc_ref[1] + w * pos_cnt
        acc_ref[2] = acc_ref[2] + w * neg_sum
        acc_ref[3] = acc_ref[3] + w * neg_cnt

    @pl.when((i == NBLK - 1) & (j == NBLK - 1))
    def _fini():
        pos_avg = acc_ref[0] / jnp.maximum(acc_ref[1], 1.0)
        neg_avg = acc_ref[2] / jnp.maximum(acc_ref[3], 1.0)
        out_ref[...] = jnp.reshape(pos_avg + neg_avg, (1, 1))


def kernel(embeddings, labels):
    emb = embeddings.astype(jnp.float32)
    lab_col = labels.astype(jnp.int32).reshape(BATCH, 1)
    lab_row = labels.astype(jnp.int32).reshape(1, BATCH)
    out = pl.pallas_call(
        _loss_body,
        grid=(NBLK, NBLK),
        in_specs=[
            pl.BlockSpec((BLK, DIM), lambda i, j: (i, 0)),
            pl.BlockSpec((BLK, DIM), lambda i, j: (j, 0)),
            pl.BlockSpec((BLK, 1), lambda i, j: (i, 0)),
            pl.BlockSpec((1, BLK), lambda i, j: (0, j)),
        ],
        out_specs=pl.BlockSpec((1, 1), lambda i, j: (0, 0)),
        out_shape=jax.ShapeDtypeStruct((1, 1), jnp.float32),
        scratch_shapes=[pltpu.SMEM((4,), jnp.float32)],
    )(emb, emb, lab_col, lab_row)
    return out[0, 0]


# speedup vs baseline: 1.3051x; 1.3051x over previous
import jax
import jax.numpy as jnp
from jax.experimental import pallas as pl
from jax.experimental.pallas import tpu as pltpu

BATCH = 4096
DIM = 128
BLK = 1024
NBLK = BATCH // BLK
NSTEPS = NBLK * (NBLK + 1) // 2
_OFFS = [0, 4, 7, 9]


def _tri_ij(t):
    i = ((t >= _OFFS[1]).astype(jnp.int32)
         + (t >= _OFFS[2]).astype(jnp.int32)
         + (t >= _OFFS[3]).astype(jnp.int32))
    off = ((t >= _OFFS[1]).astype(jnp.int32) * (_OFFS[1] - _OFFS[0])
           + (t >= _OFFS[2]).astype(jnp.int32) * (_OFFS[2] - _OFFS[1])
           + (t >= _OFFS[3]).astype(jnp.int32) * (_OFFS[3] - _OFFS[2]))
    j = t - off + i
    return i, j


def _loss_body(a_ref, b_ref, lab_i_ref, lab_j_ref, out_ref, acc_ref):
    t = pl.program_id(0)
    i, j = _tri_ij(t)

    @pl.when(t == 0)
    def _init():
        acc_ref[...] = jnp.zeros(acc_ref.shape, acc_ref.dtype)

    a = a_ref[...]
    b = b_ref[...]
    g = jax.lax.dot_general(
        a, b, dimension_numbers=(((1,), (1,)), ((), ())),
        preferred_element_type=jnp.float32)
    an = jnp.sum(a * a, axis=1, keepdims=True)
    bn = jnp.sum(b * b, axis=1)[None, :]
    sq = (an + bn) - 2.0 * g
    m = jnp.where(sq > 1e-16, sq, 1e-16)
    dist = jnp.sqrt(m)

    pos_m = lab_i_ref[...] == lab_j_ref[...]
    one = jnp.ones_like(dist)
    zero = jnp.zeros_like(dist)
    pos_f = jnp.where(pos_m, one, zero)
    omf = jnp.where(pos_m, zero, one)
    r1 = jnp.where(pos_m, dist, zero)
    tneg = 1.0 - dist
    trm = tneg > 0.0
    u = tneg * omf
    r3 = jnp.where(trm, u, zero)
    r4 = jnp.where(trm, omf, zero)

    ones_row = jnp.ones((1, BLK), jnp.float32)

    def colsum(x):
        return jax.lax.dot_general(
            ones_row, x, dimension_numbers=(((1,), (0,)), ((), ())),
            preferred_element_type=jnp.float32)

    w = jnp.where(i == j, 1.0, 2.0)
    acc_ref[0:1, :] = acc_ref[0:1, :] + w * colsum(r1)
    acc_ref[1:2, :] = acc_ref[1:2, :] + w * colsum(pos_f)
    acc_ref[2:3, :] = acc_ref[2:3, :] + w * colsum(r3)
    acc_ref[3:4, :] = acc_ref[3:4, :] + w * colsum(r4)

    @pl.when(t == NSTEPS - 1)
    def _fini():
        pos_sum = jnp.sum(acc_ref[0:1, :])
        pos_cnt = jnp.sum(acc_ref[1:2, :])
        neg_sum = jnp.sum(acc_ref[2:3, :])
        neg_cnt = jnp.sum(acc_ref[3:4, :])
        pos_avg = pos_sum / jnp.maximum(pos_cnt, 1.0)
        neg_avg = neg_sum / jnp.maximum(neg_cnt, 1.0)
        out_ref[...] = jnp.reshape(pos_avg + neg_avg, (1, 1))


def kernel(embeddings, labels):
    emb = embeddings.astype(jnp.float32)
    lab_col = labels.astype(jnp.int32).reshape(BATCH, 1)
    lab_row = labels.astype(jnp.int32).reshape(1, BATCH)
    out = pl.pallas_call(
        _loss_body,
        grid=(NSTEPS,),
        in_specs=[
            pl.BlockSpec((BLK, DIM), lambda t: (_tri_ij(t)[0], 0)),
            pl.BlockSpec((BLK, DIM), lambda t: (_tri_ij(t)[1], 0)),
            pl.BlockSpec((BLK, 1), lambda t: (_tri_ij(t)[0], 0)),
            pl.BlockSpec((1, BLK), lambda t: (0, _tri_ij(t)[1])),
        ],
        out_specs=pl.BlockSpec((1, 1), lambda t: (0, 0)),
        out_shape=jax.ShapeDtypeStruct((1, 1), jnp.float32),
        scratch_shapes=[pltpu.VMEM((8, BLK), jnp.float32)],
    )(emb, emb, lab_col, lab_row)
    return out[0, 0]


# speedup vs baseline: 1.7902x; 1.3717x over previous
import jax
import jax.numpy as jnp
from jax.experimental import pallas as pl
from jax.experimental.pallas import tpu as pltpu

BATCH = 4096
DIM = 128
BLK = 1024
NBLK = BATCH // BLK
NSTEPS = NBLK * (NBLK + 1) // 2
_OFFS = [0, 4, 7, 9]


def _tri_ij(t):
    i = ((t >= _OFFS[1]).astype(jnp.int32)
         + (t >= _OFFS[2]).astype(jnp.int32)
         + (t >= _OFFS[3]).astype(jnp.int32))
    off = ((t >= _OFFS[1]).astype(jnp.int32) * (_OFFS[1] - _OFFS[0])
           + (t >= _OFFS[2]).astype(jnp.int32) * (_OFFS[2] - _OFFS[1])
           + (t >= _OFFS[3]).astype(jnp.int32) * (_OFFS[3] - _OFFS[2]))
    j = t - off + i
    return i, j


def _loss_body(a_ref, b_ref, lab_i_ref, lab_j_ref, out_ref, acc_ref):
    t = pl.program_id(0)
    i, j = _tri_ij(t)

    @pl.when(t == 0)
    def _init():
        acc_ref[...] = jnp.zeros(acc_ref.shape, acc_ref.dtype)

    a = a_ref[...]
    b = b_ref[...]
    g = jax.lax.dot_general(
        a, b, dimension_numbers=(((1,), (1,)), ((), ())),
        preferred_element_type=jnp.float32)
    an = jnp.sum(a * a, axis=1, keepdims=True)
    bn = jnp.sum(b * b, axis=1)[None, :]
    sq = (an + bn) - 2.0 * g
    eps_i = jax.lax.bitcast_convert_type(jnp.float32(1e-16), jnp.int32)
    m = jax.lax.bitcast_convert_type(
        jnp.maximum(jax.lax.bitcast_convert_type(sq, jnp.int32), eps_i),
        jnp.float32)
    dist = m * jax.lax.rsqrt(m)

    pos_m = lab_i_ref[...] == lab_j_ref[...]
    pos_fb = jnp.where(pos_m, 1.0, 0.0).astype(jnp.bfloat16)
    omfb = jnp.bfloat16(1.0) - pos_fb
    dist_bf = dist.astype(jnp.bfloat16)
    one = jnp.ones((), jnp.bfloat16)
    zero = jnp.zeros((), jnp.bfloat16)
    tneg_bf = one - dist_bf
    trm16 = tneg_bf > zero
    s_bf = jnp.where(trm16, tneg_bf, zero)
    c_bf = jnp.where(trm16, one, zero)
    pos_f = pos_fb
    r1 = dist_bf * pos_fb
    r3 = s_bf * omfb
    r4 = c_bf * omfb

    ones_row = jnp.ones((1, BLK), jnp.bfloat16)

    def colsum(x):
        return jax.lax.dot_general(
            ones_row, x, dimension_numbers=(((1,), (0,)), ((), ())),
            preferred_element_type=jnp.float32)

    w = jnp.where(i == j, 1.0, 2.0)
    acc_ref[0:1, :] = acc_ref[0:1, :] + w * colsum(r1)
    acc_ref[1:2, :] = acc_ref[1:2, :] + w * colsum(pos_f)
    acc_ref[2:3, :] = acc_ref[2:3, :] + w * colsum(r3)
    acc_ref[3:4, :] = acc_ref[3:4, :] + w * colsum(r4)

    @pl.when(t == NSTEPS - 1)
    def _fini():
        pos_sum = jnp.sum(acc_ref[0:1, :])
        pos_cnt = jnp.sum(acc_ref[1:2, :])
        neg_sum = jnp.sum(acc_ref[2:3, :])
        neg_cnt = jnp.sum(acc_ref[3:4, :])
        pos_avg = pos_sum / jnp.maximum(pos_cnt, 1.0)
        neg_avg = neg_sum / jnp.maximum(neg_cnt, 1.0)
        out_ref[...] = jnp.reshape(pos_avg + neg_avg, (1, 1))


def kernel(embeddings, labels):
    emb = embeddings.astype(jnp.float32)
    lab_col = labels.astype(jnp.int32).reshape(BATCH, 1)
    lab_row = labels.astype(jnp.int32).reshape(1, BATCH)
    out = pl.pallas_call(
        _loss_body,
        grid=(NSTEPS,),
        in_specs=[
            pl.BlockSpec((BLK, DIM), lambda t: (_tri_ij(t)[0], 0)),
            pl.BlockSpec((BLK, DIM), lambda t: (_tri_ij(t)[1], 0)),
            pl.BlockSpec((BLK, 1), lambda t: (_tri_ij(t)[0], 0)),
            pl.BlockSpec((1, BLK), lambda t: (0, _tri_ij(t)[1])),
        ],
        out_specs=pl.BlockSpec((1, 1), lambda t: (0, 0)),
        out_shape=jax.ShapeDtypeStruct((1, 1), jnp.float32),
        scratch_shapes=[pltpu.VMEM((8, BLK), jnp.float32)],
    )(emb, emb, lab_col, lab_row)
    return out[0, 0]


# speedup vs baseline: 1.7906x; 1.0002x over previous
import jax
import jax.numpy as jnp
from jax.experimental import pallas as pl
from jax.experimental.pallas import tpu as pltpu

BATCH = 4096
DIM = 128
BLK = 1024
NBLK = BATCH // BLK
NSTEPS = NBLK * (NBLK + 1) // 2
_OFFS = [0, 4, 7, 9]


def _tri_ij(t):
    i = ((t >= _OFFS[1]).astype(jnp.int32)
         + (t >= _OFFS[2]).astype(jnp.int32)
         + (t >= _OFFS[3]).astype(jnp.int32))
    off = ((t >= _OFFS[1]).astype(jnp.int32) * (_OFFS[1] - _OFFS[0])
           + (t >= _OFFS[2]).astype(jnp.int32) * (_OFFS[2] - _OFFS[1])
           + (t >= _OFFS[3]).astype(jnp.int32) * (_OFFS[3] - _OFFS[2]))
    j = t - off + i
    return i, j


def _loss_body(a_ref, b_ref, lab_i_ref, lab_j_ref, out_ref, acc_ref):
    t = pl.program_id(0)
    i, j = _tri_ij(t)

    @pl.when(t == 0)
    def _init():
        acc_ref[...] = jnp.zeros(acc_ref.shape, acc_ref.dtype)

    a_bf = a_ref[...].astype(jnp.bfloat16)
    b_bf = b_ref[...].astype(jnp.bfloat16)
    g = jax.lax.dot_general(
        a_bf, b_bf, dimension_numbers=(((1,), (1,)), ((), ())),
        preferred_element_type=jnp.float32)
    a = a_bf.astype(jnp.float32)
    b = b_bf.astype(jnp.float32)
    an = jnp.sum(a * a, axis=1, keepdims=True)
    bn = jnp.sum(b * b, axis=1)[None, :]
    sq = (an + bn) - 2.0 * g
    eps_i = jax.lax.bitcast_convert_type(jnp.float32(1e-16), jnp.int32)
    m = jax.lax.bitcast_convert_type(
        jnp.maximum(jax.lax.bitcast_convert_type(sq, jnp.int32), eps_i),
        jnp.float32)
    dist = m * jax.lax.rsqrt(m)

    pos_m = lab_i_ref[...] == lab_j_ref[...]
    pos_fb = jnp.where(pos_m, 1.0, 0.0).astype(jnp.bfloat16)
    omfb = jnp.bfloat16(1.0) - pos_fb
    dist_bf = dist.astype(jnp.bfloat16)
    one = jnp.ones((), jnp.bfloat16)
    zero = jnp.zeros((), jnp.bfloat16)
    tneg_bf = one - dist_bf
    trm16 = tneg_bf > zero
    s_bf = jnp.where(trm16, tneg_bf, zero)
    c_bf = jnp.where(trm16, one, zero)
    pos_f = pos_fb
    r1 = dist_bf * pos_fb
    r3 = s_bf * omfb
    r4 = c_bf * omfb

    ones_row = jnp.ones((1, BLK), jnp.bfloat16)

    def colsum(x):
        return jax.lax.dot_general(
            ones_row, x, dimension_numbers=(((1,), (0,)), ((), ())),
            preferred_element_type=jnp.float32)

    w = jnp.where(i == j, 1.0, 2.0)
    acc_ref[0:1, :] = acc_ref[0:1, :] + w * colsum(r1)
    acc_ref[1:2, :] = acc_ref[1:2, :] + w * colsum(pos_f)
    acc_ref[2:3, :] = acc_ref[2:3, :] + w * colsum(r3)
    acc_ref[3:4, :] = acc_ref[3:4, :] + w * colsum(r4)

    @pl.when(t == NSTEPS - 1)
    def _fini():
        pos_sum = jnp.sum(acc_ref[0:1, :])
        pos_cnt = jnp.sum(acc_ref[1:2, :])
        neg_sum = jnp.sum(acc_ref[2:3, :])
        neg_cnt = jnp.sum(acc_ref[3:4, :])
        pos_avg = pos_sum / jnp.maximum(pos_cnt, 1.0)
        neg_avg = neg_sum / jnp.maximum(neg_cnt, 1.0)
        out_ref[...] = jnp.reshape(pos_avg + neg_avg, (1, 1))


def kernel(embeddings, labels):
    emb = embeddings.astype(jnp.float32)
    lab_col = labels.astype(jnp.int32).reshape(BATCH, 1)
    lab_row = labels.astype(jnp.int32).reshape(1, BATCH)
    out = pl.pallas_call(
        _loss_body,
        grid=(NSTEPS,),
        in_specs=[
            pl.BlockSpec((BLK, DIM), lambda t: (_tri_ij(t)[0], 0)),
            pl.BlockSpec((BLK, DIM), lambda t: (_tri_ij(t)[1], 0)),
            pl.BlockSpec((BLK, 1), lambda t: (_tri_ij(t)[0], 0)),
            pl.BlockSpec((1, BLK), lambda t: (0, _tri_ij(t)[1])),
        ],
        out_specs=pl.BlockSpec((1, 1), lambda t: (0, 0)),
        out_shape=jax.ShapeDtypeStruct((1, 1), jnp.float32),
        scratch_shapes=[pltpu.VMEM((8, BLK), jnp.float32)],
    )(emb, emb, lab_col, lab_row)
    return out[0, 0]


# speedup vs baseline: 2.0389x; 1.1386x over previous
import jax
import jax.numpy as jnp
from jax.experimental import pallas as pl
from jax.experimental.pallas import tpu as pltpu

BATCH = 4096
DIM = 128
BLK = 1024
NBLK = BATCH // BLK
NSTEPS = NBLK * (NBLK + 1) // 2
_OFFS = [0, 4, 7, 9]


def _tri_ij(t):
    i = ((t >= _OFFS[1]).astype(jnp.int32)
         + (t >= _OFFS[2]).astype(jnp.int32)
         + (t >= _OFFS[3]).astype(jnp.int32))
    off = ((t >= _OFFS[1]).astype(jnp.int32) * (_OFFS[1] - _OFFS[0])
           + (t >= _OFFS[2]).astype(jnp.int32) * (_OFFS[2] - _OFFS[1])
           + (t >= _OFFS[3]).astype(jnp.int32) * (_OFFS[3] - _OFFS[2]))
    j = t - off + i
    return i, j


def _loss_body(a_ref, b_ref, lab_i_ref, lab_j_ref, out_ref, acc_ref):
    t = pl.program_id(0)
    i, j = _tri_ij(t)

    @pl.when(t == 0)
    def _init():
        acc_ref[...] = jnp.zeros(acc_ref.shape, acc_ref.dtype)

    a = a_ref[...]
    b = b_ref[...]
    g2 = jax.lax.dot_general(
        a * (-2.0), b, dimension_numbers=(((1,), (1,)), ((), ())),
        preferred_element_type=jnp.float32)
    an = jnp.sum(a * a, axis=1, keepdims=True)
    bn = jnp.sum(b * b, axis=1)[None, :]
    sq = (an + bn) + g2
    sq_bf = sq.astype(jnp.bfloat16)
    eps = jnp.bfloat16(1e-16)
    m = jnp.where(sq_bf > eps, sq_bf, eps)
    dist_bf = m * jax.lax.rsqrt(m)

    pos_m = lab_i_ref[...] == lab_j_ref[...]
    pos_fb = jnp.where(pos_m, 1.0, 0.0).astype(jnp.bfloat16)
    omfb = jnp.bfloat16(1.0) - pos_fb
    one = jnp.ones((), jnp.bfloat16)
    zero = jnp.zeros((), jnp.bfloat16)
    tneg_bf = one - dist_bf
    trm16 = tneg_bf > zero
    s_bf = jnp.where(trm16, tneg_bf, zero)
    c_bf = jnp.where(trm16, one, zero)
    pos_f = pos_fb
    r1 = dist_bf * pos_fb
    r3 = s_bf * omfb
    r4 = c_bf * omfb

    ones_row = jnp.ones((1, BLK), jnp.bfloat16)

    def colsum(x):
        return jax.lax.dot_general(
            ones_row, x, dimension_numbers=(((1,), (0,)), ((), ())),
            preferred_element_type=jnp.float32)

    w = jnp.where(i == j, 1.0, 2.0)
    acc_ref[0:1, :] = acc_ref[0:1, :] + w * colsum(r1)
    acc_ref[1:2, :] = acc_ref[1:2, :] + w * colsum(pos_f)
    acc_ref[2:3, :] = acc_ref[2:3, :] + w * colsum(r3)
    acc_ref[3:4, :] = acc_ref[3:4, :] + w * colsum(r4)

    @pl.when(t == NSTEPS - 1)
    def _fini():
        pos_sum = jnp.sum(acc_ref[0:1, :])
        pos_cnt = jnp.sum(acc_ref[1:2, :])
        neg_sum = jnp.sum(acc_ref[2:3, :])
        neg_cnt = jnp.sum(acc_ref[3:4, :])
        pos_avg = pos_sum / jnp.maximum(pos_cnt, 1.0)
        neg_avg = neg_sum / jnp.maximum(neg_cnt, 1.0)
        out_ref[...] = jnp.reshape(pos_avg + neg_avg, (1, 1))


def kernel(embeddings, labels):
    emb = embeddings.astype(jnp.float32)
    labf = labels.astype(jnp.float32)
    lab_col = labf.reshape(BATCH, 1)
    lab_row = labf.reshape(1, BATCH)
    out = pl.pallas_call(
        _loss_body,
        grid=(NSTEPS,),
        in_specs=[
            pl.BlockSpec((BLK, DIM), lambda t: (_tri_ij(t)[0], 0)),
            pl.BlockSpec((BLK, DIM), lambda t: (_tri_ij(t)[1], 0)),
            pl.BlockSpec((BLK, 1), lambda t: (_tri_ij(t)[0], 0)),
            pl.BlockSpec((1, BLK), lambda t: (0, _tri_ij(t)[1])),
        ],
        out_specs=pl.BlockSpec((1, 1), lambda t: (0, 0)),
        out_shape=jax.ShapeDtypeStruct((1, 1), jnp.float32),
        scratch_shapes=[pltpu.VMEM((8, BLK), jnp.float32)],
    )(emb, emb, lab_col, lab_row)
    return out[0, 0]
